# baseline (speedup 1.0000x reference)
import functools
import jax
import jax.numpy as jnp
from jax import lax
from jax.experimental import pallas as pl
from jax.experimental.pallas import tpu as pltpu
from jax.experimental.pallas import tpu_sc as plsc

D_MODEL = 2048
NUM_EXPERTS = 64
TOKENS = 16384
BLOCK = 2048

_L = 16


def _matmul_t_block(x_ref, w_ref, b_ref, out_ref):
    out_ref[...] = lax.dot_general(
        w_ref[...], x_ref[...],
        (((0,), (1,)), ((), ())),
        preferred_element_type=jnp.float32,
    ) + b_ref[...]


def _logits_t(x, W, b):
    grid = TOKENS // BLOCK
    return pl.pallas_call(
        _matmul_t_block,
        grid=(grid,),
        compiler_params=pltpu.CompilerParams(
            dimension_semantics=("arbitrary",),
        ),
        in_specs=[
            pl.BlockSpec((BLOCK, D_MODEL), lambda i: (i, 0)),
            pl.BlockSpec((D_MODEL, NUM_EXPERTS), lambda i: (0, 0)),
            pl.BlockSpec((NUM_EXPERTS, 1), lambda i: (0, 0)),
        ],
        out_specs=pl.BlockSpec((NUM_EXPERTS, BLOCK), lambda i: (0, i)),
        out_shape=jax.ShapeDtypeStruct((NUM_EXPERTS, TOKENS), jnp.float32),
    )(x, W.astype(jnp.float32), b.reshape(NUM_EXPERTS, 1))


@functools.lru_cache(maxsize=None)
def _make_sc_top2():
    info = plsc.get_sparse_core_info()
    nc, ns = info.num_cores, info.num_subcores
    per_w = TOKENS // (nc * ns)

    @functools.partial(
        pl.kernel,
        mesh=plsc.VectorSubcoreMesh(core_axis_name="c", subcore_axis_name="s"),
        out_type=[
            jax.ShapeDtypeStruct((2, TOKENS), jnp.float32),
            jax.ShapeDtypeStruct((2, TOKENS), jnp.int32),
        ],
        scratch_types=[
            pltpu.VMEM((NUM_EXPERTS, per_w), jnp.float32),
            pltpu.VMEM((2, per_w), jnp.float32),
            pltpu.VMEM((2, per_w), jnp.int32),
        ],
    )
    def _sc_top2(lt_hbm, probs_hbm, idx_hbm, lv, pv, iv):
        wid = lax.axis_index("s") * nc + lax.axis_index("c")
        base = wid * per_w
        pltpu.sync_copy(lt_hbm.at[:, pl.ds(base, per_w)], lv)

        lane = lax.broadcasted_iota(jnp.int32, (_L,), 0)
        zeros = jnp.zeros((_L,), jnp.int32)
        ones = jnp.ones((_L,), jnp.int32)
        neg = jnp.full((_L,), -1e30, jnp.float32)

        def group(g, carry):
            t0 = g * _L
            m1 = lv[0, pl.ds(t0, _L)]
            i1 = zeros
            m2 = neg
            i2 = zeros
            for e in range(1, NUM_EXPERTS):
                v = lv[e, pl.ds(t0, _L)]
                e_vec = jnp.full((_L,), e, jnp.int32)
                gt1 = v > m1
                gt2 = v > m2
                m2 = jnp.where(gt1, m1, jnp.where(gt2, v, m2))
                i2 = jnp.where(gt1, i1, jnp.where(gt2, e_vec, i2))
                m1 = jnp.where(gt1, v, m1)
                i1 = jnp.where(gt1, e_vec, i1)
            e2 = jnp.exp(m2 - m1)
            p1 = 1.0 / (1.0 + e2)
            p2 = 1.0 - p1
            pv[0, pl.ds(t0, _L)] = p1
            pv[1, pl.ds(t0, _L)] = p2
            iv[0, pl.ds(t0, _L)] = i1
            iv[1, pl.ds(t0, _L)] = i2
            return carry

        lax.fori_loop(0, per_w // _L, group, 0)
        pltpu.sync_copy(pv, probs_hbm.at[:, pl.ds(base, per_w)])
        pltpu.sync_copy(iv, idx_hbm.at[:, pl.ds(base, per_w)])

    return _sc_top2


def kernel(x, W, b):
    lt = _logits_t(x, W, b)
    p_pl, i_pl = _make_sc_top2()(lt)
    return p_pl.T, i_pl.T
